# baseline (speedup 1.0000x reference)
import jax
import jax.numpy as jnp
from jax.experimental import pallas as pl
from jax.experimental.pallas import tpu as pltpu

NCH = 4


def _body(table_ref, out_ref, sem):
    b, l, d = out_ref.shape
    ch = l // NCH
    cps = []
    for c in range(NCH):
        for bi in range(b):
            cp = pltpu.make_async_copy(
                table_ref.at[pl.ds(c * ch, ch)],
                out_ref.at[bi, pl.ds(c * ch, ch)],
                sem,
            )
            cp.start()
            cps.append(cp)
    for cp in cps:
        cp.wait()


def kernel(inputs, table):
    b, l = inputs.shape
    d = table.shape[1]
    return pl.pallas_call(
        _body,
        in_specs=[pl.BlockSpec(memory_space=pltpu.MemorySpace.HBM)],
        out_specs=pl.BlockSpec(memory_space=pltpu.MemorySpace.HBM),
        out_shape=jax.ShapeDtypeStruct((b, l, d), table.dtype),
        scratch_shapes=[pltpu.SemaphoreType.DMA],
    )(table)


# speedup vs baseline: 81.1197x; 81.1197x over previous
import jax
import jax.numpy as jnp
from jax.experimental import pallas as pl
from jax.experimental.pallas import tpu as pltpu

NCH = 4


def _body(table_ref, out_ref, *scratch):
    bufs = scratch[:NCH]
    sem_in = scratch[NCH]
    sem_out = scratch[NCH + 1]
    b, l, d = out_ref.shape
    ch = l // NCH
    in_cps = []
    for c in range(NCH):
        cp = pltpu.make_async_copy(
            table_ref.at[pl.ds(c * ch, ch)], bufs[c], sem_in.at[c]
        )
        cp.start()
        in_cps.append(cp)
    out_cps = []
    for c in range(NCH):
        in_cps[c].wait()
        for bi in range(b):
            cp = pltpu.make_async_copy(
                bufs[c], out_ref.at[bi, pl.ds(c * ch, ch)], sem_out
            )
            cp.start()
            out_cps.append(cp)
    for cp in out_cps:
        cp.wait()


def kernel(inputs, table):
    b, l = inputs.shape
    d = table.shape[1]
    ch = l // NCH
    return pl.pallas_call(
        _body,
        in_specs=[pl.BlockSpec(memory_space=pltpu.MemorySpace.HBM)],
        out_specs=pl.BlockSpec(memory_space=pltpu.MemorySpace.HBM),
        out_shape=jax.ShapeDtypeStruct((b, l, d), table.dtype),
        scratch_shapes=(
            [pltpu.VMEM((ch, d), table.dtype) for _ in range(NCH)]
            + [pltpu.SemaphoreType.DMA((NCH,)), pltpu.SemaphoreType.DMA]
        ),
    )(table)
